# baseline (speedup 1.0000x reference)
import math

import jax
import jax.numpy as jnp
from jax import lax
from jax.experimental import pallas as pl
from jax.experimental.pallas import tpu as pltpu
from jax.experimental.pallas import tpu_sc as plsc

D_MODEL = 768
D_PHI = 64
NUM_BINS = 128
GAMMA = 1.0
BETA = 0.0
SEQ = 32768
M = 4096
SET_SIZE = 256
HASH_MUL = 1315423911 % NUM_BINS
HASH_ADD = 13 % NUM_BINS

L = 16
NUM_WORKERS = 32
SETS_PER_W = M // NUM_WORKERS


def _sc_hist_body(tok_hbm, idx_hbm, out_hbm, tok_v, idx_v, cnt_v):
    wid = lax.axis_index("s") * 2 + lax.axis_index("c")
    set_base = wid * SETS_PER_W
    pltpu.sync_copy(tok_hbm, tok_v)
    pltpu.sync_copy(idx_hbm.at[pl.ds(set_base, SETS_PER_W)], idx_v)

    zeros = jnp.zeros((L,), jnp.float32)

    @plsc.parallel_loop(0, SETS_PER_W, 1, unroll=4)
    def _(r):
        for j in range(NUM_BINS // L):
            cnt_v[r, pl.ds(j * L, L)] = zeros

    ones = jnp.ones((L,), jnp.float32)
    lane0 = jnp.zeros((L,), jnp.int32)

    @plsc.parallel_loop(0, SETS_PER_W, 1, unroll=2)
    def _(s):
        row = lane0 + s
        for j in range(SET_SIZE // L):
            iv = idx_v[s, pl.ds(j * L, L)]
            t = plsc.load_gather(tok_v, [iv])
            b = ((t & (NUM_BINS - 1)) * HASH_MUL + HASH_ADD) & (NUM_BINS - 1)
            plsc.addupdate_scatter(cnt_v, [row, b], ones)

    pltpu.sync_copy(cnt_v, out_hbm.at[pl.ds(set_base, SETS_PER_W)])


def _sc_histogram(token_ids, set_indices):
    mesh = plsc.VectorSubcoreMesh(core_axis_name="c", subcore_axis_name="s")
    run = pl.kernel(
        _sc_hist_body,
        out_type=jax.ShapeDtypeStruct((M, NUM_BINS), jnp.float32),
        mesh=mesh,
        scratch_types=[
            pltpu.VMEM((SEQ,), jnp.int32),
            pltpu.VMEM((SETS_PER_W, SET_SIZE), jnp.int32),
            pltpu.VMEM((SETS_PER_W, NUM_BINS), jnp.float32),
        ],
        compiler_params=pltpu.CompilerParams(needs_layout_passes=False),
    )
    return run(token_ids, set_indices)


BM_G = 256


def _geom_body(pr_ref, pc_ref, gw_ref, gb_ref, bias_ref, proj_ref):
    pr = pr_ref[...]
    pc = pc_ref[...]
    bias = -GAMMA * jnp.abs(pr - pc) + BETA
    bias_ref[...] = bias
    scale = math.exp(BETA)
    e_tile = scale * jnp.minimum(
        jnp.exp(-GAMMA * pr) * jnp.exp(GAMMA * pc),
        jnp.exp(GAMMA * pr) * jnp.exp(-GAMMA * pc))
    proj_ref[...] = jnp.dot(e_tile, gw_ref[...],
                            preferred_element_type=jnp.float32) + gb_ref[...]


def _geom(pos_row, pos_col, geom_w, geom_b):
    return pl.pallas_call(
        _geom_body,
        grid=(M // BM_G,),
        in_specs=[
            pl.BlockSpec((BM_G, 1), lambda i: (i, 0)),
            pl.BlockSpec((1, M), lambda i: (0, 0)),
            pl.BlockSpec((M, D_PHI), lambda i: (0, 0)),
            pl.BlockSpec((1, D_PHI), lambda i: (0, 0)),
        ],
        out_specs=[
            pl.BlockSpec((BM_G, M), lambda i: (i, 0)),
            pl.BlockSpec((BM_G, D_PHI), lambda i: (i, 0)),
        ],
        out_shape=[
            jax.ShapeDtypeStruct((M, M), jnp.float32),
            jax.ShapeDtypeStruct((M, D_PHI), jnp.float32),
        ],
    )(pos_row, pos_col, geom_w, geom_b)


BM_F = 1024


def _finalize_body(cnt_ref, sz_ref, pg_ref, st_ref, cw_ref, cb_ref,
                   f1g_ref, f1c_ref, fb1_ref, f2_ref, fb2_ref,
                   rf_ref, rfb_ref, rcw_ref, rcb_ref, phi_ref, desc_ref,
                   fold_w, fold_b):
    @pl.when(pl.program_id(0) == 0)
    def _():
        rf_bot = rf_ref[D_MODEL:, :]
        fold_w[...] = jnp.dot(rcw_ref[...], rf_bot,
                              preferred_element_type=jnp.float32)
        fold_b[...] = jnp.dot(rcb_ref[...], rf_bot,
                              preferred_element_type=jnp.float32) + rfb_ref[...]

    scaled = cnt_ref[...] / jnp.maximum(sz_ref[...].astype(jnp.float32), 1.0)
    pc = jnp.dot(scaled, cw_ref[...],
                 preferred_element_type=jnp.float32) + cb_ref[...]
    x = (jnp.dot(pg_ref[...], f1g_ref[...], preferred_element_type=jnp.float32)
         + jnp.dot(pc, f1c_ref[...], preferred_element_type=jnp.float32)
         + fb1_ref[...])
    h = 0.5 * x * (1.0 + lax.erf(x * 0.7071067811865476))
    phi_ref[...] = jnp.dot(h, f2_ref[...],
                           preferred_element_type=jnp.float32) + fb2_ref[...]
    desc_ref[...] = (
        jnp.dot(st_ref[...], rf_ref[:D_MODEL, :],
                preferred_element_type=jnp.float32)
        + jnp.dot(scaled, fold_w[...], preferred_element_type=jnp.float32)
        + fold_b[...])


def _finalize(counts, sizes, proj_geom, set_states, count_w, count_b,
              f1_geom, f1_cnt, fuse_b1, fuse_w2, fuse_b2,
              rfuse_w, rfuse_b, rcount_w, rcount_b):
    full = lambda r, c: pl.BlockSpec((r, c), lambda i: (0, 0))
    return pl.pallas_call(
        _finalize_body,
        grid=(M // BM_F,),
        in_specs=[
            pl.BlockSpec((BM_F, NUM_BINS), lambda i: (i, 0)),
            pl.BlockSpec((BM_F, 1), lambda i: (i, 0)),
            pl.BlockSpec((BM_F, D_PHI), lambda i: (i, 0)),
            pl.BlockSpec((BM_F, D_MODEL), lambda i: (i, 0)),
            full(NUM_BINS, D_PHI),
            full(1, D_PHI),
            full(D_PHI, D_PHI),
            full(D_PHI, D_PHI),
            full(1, D_PHI),
            full(D_PHI, D_PHI),
            full(1, D_PHI),
            full(2 * D_MODEL, D_MODEL),
            full(1, D_MODEL),
            full(NUM_BINS, D_MODEL),
            full(1, D_MODEL),
        ],
        out_specs=[
            pl.BlockSpec((BM_F, D_PHI), lambda i: (i, 0)),
            pl.BlockSpec((BM_F, D_MODEL), lambda i: (i, 0)),
        ],
        out_shape=[
            jax.ShapeDtypeStruct((M, D_PHI), jnp.float32),
            jax.ShapeDtypeStruct((M, D_MODEL), jnp.float32),
        ],
        scratch_shapes=[
            pltpu.VMEM((NUM_BINS, D_MODEL), jnp.float32),
            pltpu.VMEM((1, D_MODEL), jnp.float32),
        ],
    )(counts, sizes, proj_geom, set_states, count_w, count_b,
      f1_geom, f1_cnt, fuse_b1, fuse_w2, fuse_b2, rfuse_w, rfuse_b,
      rcount_w, rcount_b)


def kernel(token_ids, set_indices, set_sizes, set_positions, set_states,
           geom_w, geom_b, count_w, count_b, rcount_w, rcount_b,
           rfuse_w, rfuse_b, fuse_w1, fuse_b1, fuse_w2, fuse_b2):
    token_ids = token_ids.astype(jnp.int32)
    set_indices = set_indices.astype(jnp.int32)

    counts = _sc_histogram(token_ids, set_indices)

    geom_bias, proj_geom = _geom(
        set_positions.reshape(M, 1), set_positions.reshape(1, M),
        geom_w, geom_b.reshape(1, D_PHI))

    phi_attn, desc_router = _finalize(
        counts, set_sizes.reshape(M, 1), proj_geom,
        set_states, count_w, count_b.reshape(1, D_PHI),
        fuse_w1[:D_PHI], fuse_w1[D_PHI:], fuse_b1.reshape(1, D_PHI),
        fuse_w2, fuse_b2.reshape(1, D_PHI),
        rfuse_w, rfuse_b.reshape(1, D_MODEL), rcount_w,
        rcount_b.reshape(1, D_MODEL))

    return (phi_attn, desc_router, geom_bias)


# speedup vs baseline: 1.0285x; 1.0285x over previous
import math

import jax
import jax.numpy as jnp
from jax import lax
from jax.experimental import pallas as pl
from jax.experimental.pallas import tpu as pltpu
from jax.experimental.pallas import tpu_sc as plsc

D_MODEL = 768
D_PHI = 64
NUM_BINS = 128
GAMMA = 1.0
BETA = 0.0
SEQ = 32768
M = 4096
SET_SIZE = 256
HASH_MUL = 1315423911 % NUM_BINS
HASH_ADD = 13 % NUM_BINS

L = 16
NUM_WORKERS = 32
SETS_PER_W = M // NUM_WORKERS


def _sc_hist_body(tok_hbm, idx_hbm, out_hbm, tok_v, idx_v, cnt_v):
    wid = lax.axis_index("s") * 2 + lax.axis_index("c")
    set_base = wid * SETS_PER_W
    pltpu.sync_copy(tok_hbm, tok_v)
    pltpu.sync_copy(idx_hbm.at[pl.ds(set_base, SETS_PER_W)], idx_v)

    zeros = jnp.zeros((L,), jnp.float32)

    @plsc.parallel_loop(0, SETS_PER_W, 1, unroll=4)
    def _(r):
        for j in range(NUM_BINS // L):
            cnt_v[r, pl.ds(j * L, L)] = zeros

    ones = jnp.ones((L,), jnp.float32)
    lane0 = jnp.zeros((L,), jnp.int32)

    @plsc.parallel_loop(0, SETS_PER_W, 1, unroll=2)
    def _(s):
        row = lane0 + s
        for j in range(SET_SIZE // L):
            iv = idx_v[s, pl.ds(j * L, L)]
            t = plsc.load_gather(tok_v, [iv])
            b = ((t & (NUM_BINS - 1)) * HASH_MUL + HASH_ADD) & (NUM_BINS - 1)
            plsc.addupdate_scatter(cnt_v, [row, b], ones)

    pltpu.sync_copy(cnt_v, out_hbm.at[pl.ds(set_base, SETS_PER_W)])


def _sc_histogram(token_ids, set_indices):
    mesh = plsc.VectorSubcoreMesh(core_axis_name="c", subcore_axis_name="s")
    run = pl.kernel(
        _sc_hist_body,
        out_type=jax.ShapeDtypeStruct((M, NUM_BINS), jnp.float32),
        mesh=mesh,
        scratch_types=[
            pltpu.VMEM((SEQ,), jnp.int32),
            pltpu.VMEM((SETS_PER_W, SET_SIZE), jnp.int32),
            pltpu.VMEM((SETS_PER_W, NUM_BINS), jnp.float32),
        ],
        compiler_params=pltpu.CompilerParams(needs_layout_passes=False),
    )
    return run(token_ids, set_indices)


BM_G = 512


def _geom_body(pr_ref, pc_ref, gw_ref, gb_ref, bias_ref, proj_ref):
    pr = pr_ref[...]
    pc = pc_ref[...]
    bias = -GAMMA * jnp.abs(pr - pc) + BETA
    bias_ref[...] = bias
    scale = math.exp(BETA)
    e_tile = scale * jnp.minimum(
        jnp.exp(-GAMMA * pr) * jnp.exp(GAMMA * pc),
        jnp.exp(GAMMA * pr) * jnp.exp(-GAMMA * pc))
    proj_ref[...] = jnp.dot(e_tile, gw_ref[...],
                            preferred_element_type=jnp.float32) + gb_ref[...]


def _geom(pos_row, pos_col, geom_w, geom_b):
    return pl.pallas_call(
        _geom_body,
        grid=(M // BM_G,),
        in_specs=[
            pl.BlockSpec((BM_G, 1), lambda i: (i, 0)),
            pl.BlockSpec((1, M), lambda i: (0, 0)),
            pl.BlockSpec((M, D_PHI), lambda i: (0, 0)),
            pl.BlockSpec((1, D_PHI), lambda i: (0, 0)),
        ],
        out_specs=[
            pl.BlockSpec((BM_G, M), lambda i: (i, 0)),
            pl.BlockSpec((BM_G, D_PHI), lambda i: (i, 0)),
        ],
        out_shape=[
            jax.ShapeDtypeStruct((M, M), jnp.float32),
            jax.ShapeDtypeStruct((M, D_PHI), jnp.float32),
        ],
    )(pos_row, pos_col, geom_w, geom_b)


BM_F = 2048


def _finalize_body(cnt_ref, sz_ref, pg_ref, st_ref, cw_ref, cb_ref,
                   f1g_ref, f1c_ref, fb1_ref, f2_ref, fb2_ref,
                   rf_ref, rfb_ref, rcw_ref, rcb_ref, phi_ref, desc_ref,
                   fold_w, fold_b):
    @pl.when(pl.program_id(0) == 0)
    def _():
        rf_bot = rf_ref[D_MODEL:, :]
        fold_w[...] = jnp.dot(rcw_ref[...], rf_bot,
                              preferred_element_type=jnp.float32)
        fold_b[...] = jnp.dot(rcb_ref[...], rf_bot,
                              preferred_element_type=jnp.float32) + rfb_ref[...]

    scaled = cnt_ref[...] / jnp.maximum(sz_ref[...].astype(jnp.float32), 1.0)
    pc = jnp.dot(scaled, cw_ref[...],
                 preferred_element_type=jnp.float32) + cb_ref[...]
    x = (jnp.dot(pg_ref[...], f1g_ref[...], preferred_element_type=jnp.float32)
         + jnp.dot(pc, f1c_ref[...], preferred_element_type=jnp.float32)
         + fb1_ref[...])
    h = 0.5 * x * (1.0 + lax.erf(x * 0.7071067811865476))
    phi_ref[...] = jnp.dot(h, f2_ref[...],
                           preferred_element_type=jnp.float32) + fb2_ref[...]
    desc_ref[...] = (
        jnp.dot(st_ref[...], rf_ref[:D_MODEL, :],
                preferred_element_type=jnp.float32)
        + jnp.dot(scaled, fold_w[...], preferred_element_type=jnp.float32)
        + fold_b[...])


def _finalize(counts, sizes, proj_geom, set_states, count_w, count_b,
              f1_geom, f1_cnt, fuse_b1, fuse_w2, fuse_b2,
              rfuse_w, rfuse_b, rcount_w, rcount_b):
    full = lambda r, c: pl.BlockSpec((r, c), lambda i: (0, 0))
    return pl.pallas_call(
        _finalize_body,
        grid=(M // BM_F,),
        in_specs=[
            pl.BlockSpec((BM_F, NUM_BINS), lambda i: (i, 0)),
            pl.BlockSpec((BM_F, 1), lambda i: (i, 0)),
            pl.BlockSpec((BM_F, D_PHI), lambda i: (i, 0)),
            pl.BlockSpec((BM_F, D_MODEL), lambda i: (i, 0)),
            full(NUM_BINS, D_PHI),
            full(1, D_PHI),
            full(D_PHI, D_PHI),
            full(D_PHI, D_PHI),
            full(1, D_PHI),
            full(D_PHI, D_PHI),
            full(1, D_PHI),
            full(2 * D_MODEL, D_MODEL),
            full(1, D_MODEL),
            full(NUM_BINS, D_MODEL),
            full(1, D_MODEL),
        ],
        out_specs=[
            pl.BlockSpec((BM_F, D_PHI), lambda i: (i, 0)),
            pl.BlockSpec((BM_F, D_MODEL), lambda i: (i, 0)),
        ],
        out_shape=[
            jax.ShapeDtypeStruct((M, D_PHI), jnp.float32),
            jax.ShapeDtypeStruct((M, D_MODEL), jnp.float32),
        ],
        scratch_shapes=[
            pltpu.VMEM((NUM_BINS, D_MODEL), jnp.float32),
            pltpu.VMEM((1, D_MODEL), jnp.float32),
        ],
    )(counts, sizes, proj_geom, set_states, count_w, count_b,
      f1_geom, f1_cnt, fuse_b1, fuse_w2, fuse_b2, rfuse_w, rfuse_b,
      rcount_w, rcount_b)


def kernel(token_ids, set_indices, set_sizes, set_positions, set_states,
           geom_w, geom_b, count_w, count_b, rcount_w, rcount_b,
           rfuse_w, rfuse_b, fuse_w1, fuse_b1, fuse_w2, fuse_b2):
    token_ids = token_ids.astype(jnp.int32)
    set_indices = set_indices.astype(jnp.int32)

    counts = _sc_histogram(token_ids, set_indices)

    geom_bias, proj_geom = _geom(
        set_positions.reshape(M, 1), set_positions.reshape(1, M),
        geom_w, geom_b.reshape(1, D_PHI))

    phi_attn, desc_router = _finalize(
        counts, set_sizes.reshape(M, 1), proj_geom,
        set_states, count_w, count_b.reshape(1, D_PHI),
        fuse_w1[:D_PHI], fuse_w1[D_PHI:], fuse_b1.reshape(1, D_PHI),
        fuse_w2, fuse_b2.reshape(1, D_PHI),
        rfuse_w, rfuse_b.reshape(1, D_MODEL), rcount_w,
        rcount_b.reshape(1, D_MODEL))

    return (phi_attn, desc_router, geom_bias)
